# baseline (device time: 63050 ns/iter reference)
import jax
import jax.numpy as jnp
from jax import lax
from jax.experimental import pallas as pl
from jax.experimental.pallas import tpu as pltpu

N_DEV = 8
EXCH = ((1, 2), (2, 4), (1, 4), (4, 8), (2, 8), (1, 8))
N_EXCH = len(EXCH)
N_CHUNK = 1


def kernel(x):
    m_per, n = x.shape
    n2 = n // 2
    mc = m_per // N_CHUNK
    log_m = m_per.bit_length() - 1

    def body(x_ref, out_ref, sbuf_ref, recv_ref, send_sems, recv_sems):
        p = lax.axis_index("i")
        row1 = lax.broadcasted_iota(jnp.int32, (m_per, 1), 0)

        def local_stage(v, j, k):
            bit = (row1 & j) != 0
            if j >= 8:
                cols = v.shape[1]
                partner = pltpu.roll(
                    v.reshape(m_per // (2 * j), 2 * j, cols), j, 1
                ).reshape(m_per, cols)
            else:
                up = pltpu.roll(v, m_per - j, 0)
                down = pltpu.roll(v, j, 0)
                partner = jnp.where(bit, down, up)
            if k < m_per:
                take_max = bit ^ ((row1 & k) != 0)
            else:
                take_max = bit ^ ((p & (k // m_per)) != 0)
            return jnp.where(take_max, jnp.maximum(v, partner),
                             jnp.minimum(v, partner))

        def local_sort(v):
            for kexp in range(1, log_m + 1):
                k = 1 << kexp
                j = k // 2
                while j >= 1:
                    v = local_stage(v, j, k)
                    j //= 2
            return v

        def local_merge(v, k):
            j = m_per // 2
            while j >= 1:
                v = local_stage(v, j, k)
                j //= 2
            return v

        chunks = {}
        bchunks = {}
        rdmas = {}

        def split(h, v):
            for r in range(N_CHUNK):
                chunks[(h, r)] = v[r * mc:(r + 1) * mc, :]

        def join(h):
            return jnp.concatenate(
                [chunks[(h, r)] for r in range(N_CHUNK)], axis=0
            )

        def send(h, s, r):
            b = chunks[(h, r)].astype(jnp.bfloat16)
            bchunks[(h, r)] = b
            sbuf_ref[h, r * mc:(r + 1) * mc, :] = b
            rdma = pltpu.make_async_remote_copy(
                src_ref=sbuf_ref.at[h, pl.ds(r * mc, mc)],
                dst_ref=recv_ref.at[h, s, pl.ds(r * mc, mc)],
                send_sem=send_sems.at[h, s, r],
                recv_sem=recv_sems.at[h, s, r],
                device_id=(p ^ EXCH[s][0],),
                device_id_type=pl.DeviceIdType.MESH,
            )
            rdma.start()
            rdmas[(h, s, r)] = rdma

        def finish(h, s, r):
            rdmas[(h, s, r)].wait()
            theirs_b = recv_ref[h, s, r * mc:(r + 1) * mc, :]
            j_dev, k_dev = EXCH[s]
            take_max = ((p & j_dev) != 0) ^ ((p & k_dev) != 0)
            mine = chunks[(h, r)]
            theirs = theirs_b.astype(jnp.float32)
            cmp = bchunks[(h, r)] < theirs_b
            min_res = jnp.where(cmp, mine, theirs)
            max_res = jnp.where(cmp, theirs, mine)
            chunks[(h, r)] = jnp.where(take_max, max_res, min_res)

        def merge_boundary(h, s_prev, s_next, k):
            for r in range(N_CHUNK):
                finish(h, s_prev, r)
            split(h, local_merge(join(h), k))
            if s_next is not None:
                for r in range(N_CHUNK):
                    send(h, s_next, r)

        split(0, local_sort(x_ref[:, 0:n2]))

        barrier_sem = pltpu.get_barrier_semaphore()
        for m in (1, 2, 4):
            pl.semaphore_signal(
                barrier_sem, inc=1,
                device_id=(p ^ m,), device_id_type=pl.DeviceIdType.MESH,
            )
        pl.semaphore_wait(barrier_sem, 3)

        for r in range(N_CHUNK):
            send(0, 0, r)
        split(1, local_sort(x_ref[:, n2:n]))
        for r in range(N_CHUNK):
            send(1, 0, r)

        def fin(h, s):
            for r in range(N_CHUNK):
                finish(h, s, r)

        def snd(h, s):
            for r in range(N_CHUNK):
                send(h, s, r)

        fin(0, 0)
        split(0, local_merge(join(0), 2 * m_per))
        snd(0, 1)
        fin(1, 0)
        split(1, local_merge(join(1), 2 * m_per))
        fin(0, 1)
        snd(0, 2)
        snd(1, 1)
        fin(0, 2)
        split(0, local_merge(join(0), 4 * m_per))
        snd(0, 3)
        fin(1, 1)
        snd(1, 2)
        fin(0, 3)
        snd(0, 4)
        fin(1, 2)
        split(1, local_merge(join(1), 4 * m_per))
        snd(1, 3)
        fin(0, 4)
        snd(0, 5)
        fin(1, 3)
        snd(1, 4)
        fin(0, 5)
        out_ref[:, 0:n2] = local_merge(join(0), 8 * m_per)
        fin(1, 4)
        snd(1, 5)
        fin(1, 5)
        out_ref[:, n2:n] = local_merge(join(1), 8 * m_per)

    return pl.pallas_call(
        body,
        out_shape=jax.ShapeDtypeStruct((m_per, n), x.dtype),
        in_specs=[pl.BlockSpec(memory_space=pltpu.VMEM)],
        out_specs=pl.BlockSpec(memory_space=pltpu.VMEM),
        scratch_shapes=[
            pltpu.VMEM((2, m_per, n2), jnp.bfloat16),
            pltpu.VMEM((2, N_EXCH, m_per, n2), jnp.bfloat16),
            pltpu.SemaphoreType.DMA((2, N_EXCH, N_CHUNK)),
            pltpu.SemaphoreType.DMA((2, N_EXCH, N_CHUNK)),
        ],
        compiler_params=pltpu.CompilerParams(
            collective_id=0,
            vmem_limit_bytes=100 * 1024 * 1024,
        ),
    )(x)


# device time: 59552 ns/iter; 1.0587x vs baseline; 1.0587x over previous
import jax
import jax.numpy as jnp
from jax import lax
from jax.experimental import pallas as pl
from jax.experimental.pallas import tpu as pltpu

N_DEV = 8
EXCH = ((1, 2), (2, 4), (1, 4), (4, 8), (2, 8), (1, 8))
N_EXCH = len(EXCH)
N_CHUNK = 1


def kernel(x):
    m_per, n = x.shape
    n2 = n // 2
    mc = m_per // N_CHUNK
    log_m = m_per.bit_length() - 1

    def body(x_ref, out_ref, sbuf_ref, recv_ref, send_sems, recv_sems):
        p = lax.axis_index("i")
        row1 = lax.broadcasted_iota(jnp.int32, (m_per, 1), 0)

        def local_stage(v, j, k):
            bit = (row1 & j) != 0
            if j >= 8:
                cols = v.shape[1]
                partner = pltpu.roll(
                    v.reshape(m_per // (2 * j), 2 * j, cols), j, 1
                ).reshape(m_per, cols)
            else:
                up = pltpu.roll(v, m_per - j, 0)
                down = pltpu.roll(v, j, 0)
                partner = jnp.where(bit, down, up)
            if k < m_per:
                take_max = bit ^ ((row1 & k) != 0)
            else:
                take_max = bit ^ ((p & (k // m_per)) != 0)
            return jnp.where(take_max, jnp.maximum(v, partner),
                             jnp.minimum(v, partner))

        def local_sort(v):
            for kexp in range(1, log_m + 1):
                k = 1 << kexp
                j = k // 2
                while j >= 1:
                    v = local_stage(v, j, k)
                    j //= 2
            return v

        def local_merge(v, k):
            j = m_per // 2
            while j >= 1:
                v = local_stage(v, j, k)
                j //= 2
            return v

        chunks = {}
        bchunks = {}
        rdmas = {}

        def split(h, v):
            for r in range(N_CHUNK):
                chunks[(h, r)] = v[r * mc:(r + 1) * mc, :]

        def join(h):
            return jnp.concatenate(
                [chunks[(h, r)] for r in range(N_CHUNK)], axis=0
            )

        def send(h, s, r):
            b = chunks[(h, r)].astype(jnp.bfloat16)
            bchunks[(h, r)] = b
            sbuf_ref[h, r * mc:(r + 1) * mc, :] = b
            rdma = pltpu.make_async_remote_copy(
                src_ref=sbuf_ref.at[h, pl.ds(r * mc, mc)],
                dst_ref=recv_ref.at[h, s, pl.ds(r * mc, mc)],
                send_sem=send_sems.at[h, s, r],
                recv_sem=recv_sems.at[h, s, r],
                device_id=(p ^ EXCH[s][0],),
                device_id_type=pl.DeviceIdType.MESH,
            )
            rdma.start()
            rdmas[(h, s, r)] = rdma

        def finish(h, s, r):
            rdmas[(h, s, r)].wait()
            theirs_b = recv_ref[h, s, r * mc:(r + 1) * mc, :]
            j_dev, k_dev = EXCH[s]
            take_max = ((p & j_dev) != 0) ^ ((p & k_dev) != 0)
            mine = chunks[(h, r)]
            theirs = theirs_b.astype(jnp.float32)
            cmp = bchunks[(h, r)] < theirs_b
            min_res = jnp.where(cmp, mine, theirs)
            max_res = jnp.where(cmp, theirs, mine)
            chunks[(h, r)] = jnp.where(take_max, max_res, min_res)

        def merge_boundary(h, s_prev, s_next, k):
            for r in range(N_CHUNK):
                finish(h, s_prev, r)
            split(h, local_merge(join(h), k))
            if s_next is not None:
                for r in range(N_CHUNK):
                    send(h, s_next, r)

        split(0, local_sort(x_ref[:, 0:n2]))

        barrier_sem = pltpu.get_barrier_semaphore()
        for m in (1, 2, 4):
            pl.semaphore_signal(
                barrier_sem, inc=1,
                device_id=(p ^ m,), device_id_type=pl.DeviceIdType.MESH,
            )
        pl.semaphore_wait(barrier_sem, 3)

        for r in range(N_CHUNK):
            send(0, 0, r)
        split(1, local_sort(x_ref[:, n2:n]))
        for r in range(N_CHUNK):
            send(1, 0, r)

        for h in (0, 1):
            merge_boundary(h, 0, 1, 2 * m_per)
        for r in range(N_CHUNK):
            for h in (0, 1):
                finish(h, 1, r)
                send(h, 2, r)
        for h in (0, 1):
            merge_boundary(h, 2, 3, 4 * m_per)
        for s in (3, 4):
            for r in range(N_CHUNK):
                for h in (0, 1):
                    finish(h, s, r)
                    send(h, s + 1, r)
        for h in (0, 1):
            for r in range(N_CHUNK):
                finish(h, 5, r)
            out_ref[:, h * n2:(h + 1) * n2] = local_merge(join(h), 8 * m_per)

    return pl.pallas_call(
        body,
        out_shape=jax.ShapeDtypeStruct((m_per, n), x.dtype),
        in_specs=[pl.BlockSpec(memory_space=pltpu.VMEM)],
        out_specs=pl.BlockSpec(memory_space=pltpu.VMEM),
        scratch_shapes=[
            pltpu.VMEM((2, m_per, n2), jnp.bfloat16),
            pltpu.VMEM((2, N_EXCH, m_per, n2), jnp.bfloat16),
            pltpu.SemaphoreType.DMA((2, N_EXCH, N_CHUNK)),
            pltpu.SemaphoreType.DMA((2, N_EXCH, N_CHUNK)),
        ],
        compiler_params=pltpu.CompilerParams(
            collective_id=0,
            vmem_limit_bytes=100 * 1024 * 1024,
        ),
    )(x)


# device time: 59234 ns/iter; 1.0644x vs baseline; 1.0054x over previous
import jax
import jax.numpy as jnp
from jax import lax
from jax.experimental import pallas as pl
from jax.experimental.pallas import tpu as pltpu

N_DEV = 8
EXCH = ((1, 2), (2, 4), (1, 4), (4, 8), (2, 8), (1, 8))
N_EXCH = len(EXCH)
N_CHUNK = 1


def kernel(x):
    m_per, n = x.shape
    n2 = n // 2
    mc = m_per // N_CHUNK
    log_m = m_per.bit_length() - 1

    def body(x_ref, out_ref, sbuf_ref, recv_ref, send_sems, recv_sems):
        p = lax.axis_index("i")
        row1 = lax.broadcasted_iota(jnp.int32, (m_per, 1), 0)

        def local_stage(v, j, k):
            bit = (row1 & j) != 0
            if j >= 8:
                cols = v.shape[1]
                partner = pltpu.roll(
                    v.reshape(m_per // (2 * j), 2 * j, cols), j, 1
                ).reshape(m_per, cols)
            else:
                up = pltpu.roll(v, m_per - j, 0)
                down = pltpu.roll(v, j, 0)
                partner = jnp.where(bit, down, up)
            if k < m_per:
                take_max = bit ^ ((row1 & k) != 0)
            else:
                take_max = bit ^ ((p & (k // m_per)) != 0)
            return jnp.where(take_max, jnp.maximum(v, partner),
                             jnp.minimum(v, partner))

        def local_sort(v):
            for kexp in range(1, log_m + 1):
                k = 1 << kexp
                j = k // 2
                while j >= 1:
                    v = local_stage(v, j, k)
                    j //= 2
            return v

        def local_merge(v, k):
            j = m_per // 2
            while j >= 1:
                v = local_stage(v, j, k)
                j //= 2
            return v

        chunks = {}
        bchunks = {}
        rdmas = {}

        def split(h, v):
            for r in range(N_CHUNK):
                chunks[(h, r)] = v[r * mc:(r + 1) * mc, :]

        def join(h):
            return jnp.concatenate(
                [chunks[(h, r)] for r in range(N_CHUNK)], axis=0
            )

        def send(h, s, r):
            b = chunks[(h, r)].astype(jnp.bfloat16)
            bchunks[(h, r)] = b
            sbuf_ref[h, r * mc:(r + 1) * mc, :] = b
            rdma = pltpu.make_async_remote_copy(
                src_ref=sbuf_ref.at[h, pl.ds(r * mc, mc)],
                dst_ref=recv_ref.at[h, s, pl.ds(r * mc, mc)],
                send_sem=send_sems.at[h, s, r],
                recv_sem=recv_sems.at[h, s, r],
                device_id=(p ^ EXCH[s][0],),
                device_id_type=pl.DeviceIdType.MESH,
            )
            rdma.start()
            rdmas[(h, s, r)] = rdma

        def finish(h, s, r):
            rdmas[(h, s, r)].wait()
            theirs_b = recv_ref[h, s, r * mc:(r + 1) * mc, :]
            j_dev, k_dev = EXCH[s]
            take_max = ((p & j_dev) != 0) ^ ((p & k_dev) != 0)
            mine = chunks[(h, r)]
            theirs = theirs_b.astype(jnp.float32)
            cmp = bchunks[(h, r)] < theirs_b
            chunks[(h, r)] = jnp.where(cmp ^ take_max, mine, theirs)

        def merge_boundary(h, s_prev, s_next, k):
            for r in range(N_CHUNK):
                finish(h, s_prev, r)
            split(h, local_merge(join(h), k))
            if s_next is not None:
                for r in range(N_CHUNK):
                    send(h, s_next, r)

        split(0, local_sort(x_ref[:, 0:n2]))

        barrier_sem = pltpu.get_barrier_semaphore()
        for m in (1, 2, 4):
            pl.semaphore_signal(
                barrier_sem, inc=1,
                device_id=(p ^ m,), device_id_type=pl.DeviceIdType.MESH,
            )
        pl.semaphore_wait(barrier_sem, 3)

        for r in range(N_CHUNK):
            send(0, 0, r)
        split(1, local_sort(x_ref[:, n2:n]))
        for r in range(N_CHUNK):
            send(1, 0, r)

        for h in (0, 1):
            merge_boundary(h, 0, 1, 2 * m_per)
        for r in range(N_CHUNK):
            for h in (0, 1):
                finish(h, 1, r)
                send(h, 2, r)
        for h in (0, 1):
            merge_boundary(h, 2, 3, 4 * m_per)
        for s in (3, 4):
            for r in range(N_CHUNK):
                for h in (0, 1):
                    finish(h, s, r)
                    send(h, s + 1, r)
        for h in (0, 1):
            for r in range(N_CHUNK):
                finish(h, 5, r)
            out_ref[:, h * n2:(h + 1) * n2] = local_merge(join(h), 8 * m_per)

    return pl.pallas_call(
        body,
        out_shape=jax.ShapeDtypeStruct((m_per, n), x.dtype),
        in_specs=[pl.BlockSpec(memory_space=pltpu.VMEM)],
        out_specs=pl.BlockSpec(memory_space=pltpu.VMEM),
        scratch_shapes=[
            pltpu.VMEM((2, m_per, n2), jnp.bfloat16),
            pltpu.VMEM((2, N_EXCH, m_per, n2), jnp.bfloat16),
            pltpu.SemaphoreType.DMA((2, N_EXCH, N_CHUNK)),
            pltpu.SemaphoreType.DMA((2, N_EXCH, N_CHUNK)),
        ],
        compiler_params=pltpu.CompilerParams(
            collective_id=0,
            vmem_limit_bytes=100 * 1024 * 1024,
        ),
    )(x)
